# baseline (speedup 1.0000x reference)
import functools

import jax
import jax.numpy as jnp
from jax import lax
from jax.experimental import pallas as pl
from jax.experimental.pallas import tpu as pltpu
from jax.experimental.pallas import tpu_sc as plsc

N = 10000
E = 320000
D_IN = 128
D_EDGE = 16
HID = 128
DEPTH = 3
NUM_GRAPHS = 64

NC = 2
NS = 16
BLK = 64
RWA = 216
RWB = 108
ROWS = NS * (RWA + RWB)
EPAD = ROWS * BLK
N2 = 10240
NPAD = 10112
STRIPE = NPAD // NS


def _proj_h_body(x_ref, w_ref, b_ref, o_ref):
    o_ref[...] = jnp.maximum(
        jnp.dot(x_ref[...], w_ref[...], preferred_element_type=jnp.float32)
        + b_ref[...], 0.0)


def _proj_h(x, w, b):
    blk = 512
    return pl.pallas_call(
        _proj_h_body,
        grid=(N2 // blk,),
        in_specs=[
            pl.BlockSpec((blk, D_IN), lambda i: (i, 0)),
            pl.BlockSpec((D_IN, HID), lambda i: (0, 0)),
            pl.BlockSpec((1, HID), lambda i: (0, 0)),
        ],
        out_specs=pl.BlockSpec((blk, HID), lambda i: (i, 0)),
        out_shape=jax.ShapeDtypeStruct((N2, HID), jnp.float32),
    )(x, w, b)


def _proj_e_body(ea_ref, w_ref, b_ref, o_ref):
    o_ref[...] = (
        jnp.dot(ea_ref[...], w_ref[...], preferred_element_type=jnp.float32)
        + b_ref[...])


def _proj_e(ea8, wk, bk):
    blk = 256
    rows8 = EPAD // 8
    return pl.pallas_call(
        _proj_e_body,
        grid=(rows8 // blk,),
        in_specs=[
            pl.BlockSpec((blk, 8 * D_EDGE), lambda i: (i, 0)),
            pl.BlockSpec((8 * D_EDGE, 8 * HID), lambda i: (0, 0)),
            pl.BlockSpec((1, 8 * HID), lambda i: (0, 0)),
        ],
        out_specs=pl.BlockSpec((blk, 8 * HID), lambda i: (i, 0)),
        out_shape=jax.ShapeDtypeStruct((rows8, 8 * HID), jnp.float32),
    )(ea8, wk, bk)


def _mlp_core(h, a0, a1, w1_ref, b1_ref, w2_ref, b2_ref):
    z = h + a0 + a1
    z = jnp.maximum(
        jnp.dot(z, w1_ref[...], preferred_element_type=jnp.float32)
        + b1_ref[...], 0.0)
    return jnp.dot(z, w2_ref[...], preferred_element_type=jnp.float32) \
        + b2_ref[...]


def _mlp_body(h_ref, a0_ref, a1_ref, w1_ref, b1_ref, w2_ref, b2_ref, o_ref):
    z = _mlp_core(h_ref[...], a0_ref[0], a1_ref[0],
                  w1_ref, b1_ref, w2_ref, b2_ref)
    o_ref[...] = jnp.maximum(z, 0.0)


def _mlp(h, agg, w1, b1, w2, b2):
    blk = 1000
    return pl.pallas_call(
        _mlp_body,
        grid=(N // blk,),
        in_specs=[
            pl.BlockSpec((blk, HID), lambda i: (i, 0)),
            pl.BlockSpec((1, blk, HID), lambda i: (0, i, 0)),
            pl.BlockSpec((1, blk, HID), lambda i: (1, i, 0)),
            pl.BlockSpec((HID, HID), lambda i: (0, 0)),
            pl.BlockSpec((1, HID), lambda i: (0, 0)),
            pl.BlockSpec((HID, HID), lambda i: (0, 0)),
            pl.BlockSpec((1, HID), lambda i: (0, 0)),
        ],
        out_specs=pl.BlockSpec((blk, HID), lambda i: (i, 0)),
        out_shape=jax.ShapeDtypeStruct((N2, HID), jnp.float32),
    )(h, agg, agg, w1, b1, w2, b2)


def _mlp_pool_body(h_ref, a0_ref, a1_ref, w1_ref, b1_ref, w2_ref, b2_ref,
                   batch_ref, o_ref):
    i = pl.program_id(0)
    z = _mlp_core(h_ref[...], a0_ref[0], a1_ref[0],
                  w1_ref, b1_ref, w2_ref, b2_ref)
    gids = lax.broadcasted_iota(jnp.int32, (z.shape[0], NUM_GRAPHS), 1)
    onehot = (batch_ref[...] == gids).astype(jnp.float32)
    part = lax.dot_general(onehot, z, (((0,), (0,)), ((), ())),
                           preferred_element_type=jnp.float32)

    @pl.when(i == 0)
    def _():
        o_ref[...] = jnp.zeros_like(o_ref)

    o_ref[...] += part


def _mlp_pool(h, agg, w1, b1, w2, b2, batch2d):
    blk = 1000
    return pl.pallas_call(
        _mlp_pool_body,
        grid=(N // blk,),
        in_specs=[
            pl.BlockSpec((blk, HID), lambda i: (i, 0)),
            pl.BlockSpec((1, blk, HID), lambda i: (0, i, 0)),
            pl.BlockSpec((1, blk, HID), lambda i: (1, i, 0)),
            pl.BlockSpec((HID, HID), lambda i: (0, 0)),
            pl.BlockSpec((1, HID), lambda i: (0, 0)),
            pl.BlockSpec((HID, HID), lambda i: (0, 0)),
            pl.BlockSpec((1, HID), lambda i: (0, 0)),
            pl.BlockSpec((blk, 1), lambda i: (i, 0)),
        ],
        out_specs=pl.BlockSpec((NUM_GRAPHS, HID), lambda i: (0, 0)),
        out_shape=jax.ShapeDtypeStruct((NUM_GRAPHS, HID), jnp.float32),
    )(h, agg, agg, w1, b1, w2, b2, batch2d)


def _sc_agg_body(h_hbm, e_hbm, ei_hbm, out_hbm,
                 srcv, dstv, hrows, erows, aggsh,
                 si0, si1, si2, sj0, sj1, sj2, sg0, sg1, sg2, sg3,
                 se0, se1, ss0, ss1, ss2, ss3):
    si = (si0, si1, si2)
    sj = (sj0, sj1, sj2)
    sg = (sg0, sg1, sg2, sg3)
    se = (se0, se1)
    ss = (ss0, ss1, ss2, ss3)
    c = lax.axis_index("c")
    s = lax.axis_index("s")
    base = s * (RWA + RWB) + jnp.where(c == 0, 0, RWA)
    rw = jnp.where(c == 0, RWA, RWB)
    n_outer = jnp.where(c == 0, RWA // 12, RWB // 12)

    def zrow(i, _):
        for j in range(HID // 16):
            hrows[0, i, pl.ds(j * 16, 16)] = jnp.zeros((16,), jnp.float32)
        return 0
    lax.fori_loop(0, BLK, zrow, 0)
    for k in range(STRIPE // BLK):
        pltpu.sync_copy(hrows.at[0],
                        aggsh.at[pl.ds(s * STRIPE + k * BLK, BLK)])
    rem_rows = STRIPE - (STRIPE // BLK) * BLK
    if rem_rows:
        pltpu.sync_copy(
            hrows.at[0, pl.ds(0, rem_rows)],
            aggsh.at[pl.ds(s * STRIPE + (STRIPE // BLK) * BLK, rem_rows)])
    plsc.subcore_barrier()

    def issue_src(r, slot):
        pltpu.async_copy(ei_hbm.at[0, base + r], srcv.at[slot], si[slot])

    def wait_src(r, slot):
        pltpu.make_async_copy(ei_hbm.at[0, base + r], srcv.at[slot],
                              si[slot]).wait()

    def issue_dst(r, slot):
        pltpu.async_copy(ei_hbm.at[1, base + r], dstv.at[slot], sj[slot])

    def wait_dst(r, slot):
        pltpu.make_async_copy(ei_hbm.at[1, base + r], dstv.at[slot],
                              sj[slot]).wait()

    def issue_gather(r, gslot, sslot):
        pltpu.async_copy(h_hbm.at[srcv.at[sslot]], hrows.at[gslot],
                         sg[gslot])

    def wait_gather(r, gslot, sslot):
        pltpu.make_async_copy(h_hbm.at[srcv.at[sslot]], hrows.at[gslot],
                              sg[gslot]).wait()

    def issue_e(r, slot):
        pltpu.async_copy(e_hbm.at[pl.ds((base + r) * 8, 8)], erows.at[slot],
                         se[slot])

    def wait_e(r, slot):
        pltpu.make_async_copy(e_hbm.at[pl.ds((base + r) * 8, 8)],
                              erows.at[slot], se[slot]).wait()

    def issue_scatter(r, gslot, jslot):
        pltpu.async_copy(hrows.at[gslot], aggsh.at[dstv.at[jslot]],
                         ss[gslot], add=True)

    def wait_scatter(r, gslot, jslot):
        pltpu.make_async_copy(hrows.at[gslot], aggsh.at[dstv.at[jslot]],
                              ss[gslot]).wait()

    issue_src(0, 0)
    issue_src(1, 1)
    issue_src(2, 2)
    issue_dst(0, 0)
    wait_src(0, 0)
    issue_gather(0, 0, 0)
    issue_e(0, 0)
    wait_src(1, 1)
    issue_gather(1, 1, 1)

    def outer(it, _):
        r0 = it * 12
        for b in range(12):
            r = r0 + b
            b4 = b % 4
            b3 = b % 3
            b2 = b % 2

            @pl.when(r >= 2)
            def _():
                wait_scatter(r - 2, (b + 2) % 4, (b + 1) % 3)

            @pl.when(r + 2 < rw)
            def _():
                wait_src(r + 2, (b + 2) % 3)
                issue_gather(r + 2, (b + 2) % 4, (b + 2) % 3)

            @pl.when(r + 1 < rw)
            def _():
                issue_e(r + 1, (b + 1) % 2)
                issue_dst(r + 1, (b + 1) % 3)

            wait_gather(r, b4, b3)
            wait_e(r, b2)

            @pl.when(r + 3 < rw)
            def _():
                issue_src(r + 3, b3)

            def msg_blk(i8, _):
                for jh in range(8):
                    for g in range(HID // 16):
                        sl = pl.ds(g * 16, 16)
                        esl = pl.ds(jh * HID + g * 16, 16)
                        hrows[b4, i8 * 8 + jh, sl] = jnp.maximum(
                            hrows[b4, i8 * 8 + jh, sl]
                            + erows[b2, i8, esl], 0.0)
                return 0
            lax.fori_loop(0, BLK // 8, msg_blk, 0)

            wait_dst(r, b3)
            issue_scatter(r, b4, b3)
        return 0

    lax.fori_loop(0, n_outer, outer, 0)
    wait_scatter(rw - 2, 2, 1)
    wait_scatter(rw - 1, 3, 2)

    plsc.subcore_barrier()
    pltpu.sync_copy(aggsh.at[pl.ds(s * STRIPE, STRIPE)],
                    out_hbm.at[c, pl.ds(s * STRIPE, STRIPE)])


_sc_agg = functools.partial(
    pl.kernel,
    out_type=jax.ShapeDtypeStruct((NC, NPAD, HID), jnp.float32),
    mesh=plsc.VectorSubcoreMesh(core_axis_name="c", subcore_axis_name="s"),
    scratch_types=[
        pltpu.VMEM((3, BLK), jnp.int32),
        pltpu.VMEM((3, BLK), jnp.int32),
        pltpu.VMEM((4, BLK, HID), jnp.float32),
        pltpu.VMEM((2, 8, 8 * HID), jnp.float32),
        pltpu.VMEM_SHARED((NPAD, HID), jnp.float32),
    ] + [pltpu.SemaphoreType.DMA] * 16,
)(_sc_agg_body)


def kernel(x, edge_attr, edge_index, batch,
           W_node, b_node, W_edge, b_edge, W1, b1, W2, b2):
    pad = EPAD - E
    ei = jnp.pad(edge_index, ((0, 0), (0, pad)), constant_values=N)
    ei = ei.reshape(2, ROWS, BLK)
    ea8 = jnp.pad(edge_attr, ((0, pad), (0, 0))).reshape(EPAD // 8,
                                                         8 * D_EDGE)
    wk = jnp.kron(jnp.eye(8, dtype=jnp.float32), W_edge)
    bk = jnp.tile(b_edge, 8).reshape(1, 8 * HID)
    batch2d = batch.reshape(N, 1)

    h = _proj_h(x, W_node, b_node.reshape(1, HID))
    e = _proj_e(ea8, wk, bk)

    for i in range(DEPTH):
        agg = _sc_agg(h, e, ei)
        if i < DEPTH - 1:
            h = _mlp(h, agg, W1[i], b1[i].reshape(1, HID),
                     W2[i], b2[i].reshape(1, HID))
        else:
            out = _mlp_pool(h, agg, W1[i], b1[i].reshape(1, HID),
                            W2[i], b2[i].reshape(1, HID), batch2d)
    return out
